# baseline (speedup 1.0000x reference)
import functools

import jax
import jax.numpy as jnp
from jax import lax
from jax.experimental import pallas as pl
from jax.experimental.pallas import tpu as pltpu
from jax.experimental.pallas import tpu_sc as plsc

B0 = 2
NR = 4096
DK = 2048
N = B0 * NR * DK
KTH = int(0.9999 * N)
KP = N - KTH + 1
NC = 2
NS = 16
L = 16
RPT = NR // NS
BA = 1 << 15
BB = 1 << 16

_mesh = plsc.VectorSubcoreMesh(core_axis_name="c", subcore_axis_name="s")
_params = pltpu.CompilerParams(needs_layout_passes=False)


def _splat(x):
    return jnp.broadcast_to(x, (L,))


def _zero2d(ref, nbins):
    z = jnp.zeros((L,), jnp.int32)

    @plsc.parallel_loop(0, nbins // L, unroll=8)
    def _(t):
        ref[t // 8, pl.ds((t % 8) * L, L)] = z


def _fill_idx(idx, ngrp):
    iota = lax.iota(jnp.int32, L)

    def b(t, _):
        r = t // 8
        k = t % 8
        idx[r, pl.ds(k * L, L)] = r * 128 + k * L + iota
        return 0

    lax.fori_loop(0, ngrp * 8, b, 0)


def _db_pass(x_hbm, c, s, rch, buf0, buf1, sem0, sem1, process):
    r0 = s * RPT
    nch = RPT // rch

    def start(ci, buf, sem):
        pltpu.async_copy(x_hbm.at[c, pl.ds(r0 + ci * rch, rch), :], buf, sem)

    def wait(ci, buf, sem):
        pltpu.make_async_copy(
            x_hbm.at[c, pl.ds(r0 + ci * rch, rch), :], buf, sem).wait()

    start(0, buf0, sem0)

    def body(i, _):
        ci0 = 2 * i
        start(ci0 + 1, buf1, sem1)
        wait(ci0, buf0, sem0)
        process(buf0)

        @pl.when(ci0 + 2 < nch)
        def _():
            start(ci0 + 2, buf0, sem0)

        wait(ci0 + 1, buf1, sem1)
        process(buf1)
        return 0

    lax.fori_loop(0, nch // 2, body, 0)


def _scan_block(ba0, ba1, blksz, base_bin, cum0, rank_s):
    z = jnp.int32(0)

    def ch_body(i, carry):
        cum2, found, jch, cum2_at = carry
        t = blksz // L - 1 - i
        sl = pl.ds(t * L, L)
        ct = jnp.sum(ba0[sl] + ba1[sl])
        hit = jnp.logical_and(found == 0, cum2 + ct >= rank_s)
        jch = jnp.where(hit, t, jch)
        cum2_at = jnp.where(hit, cum2, cum2_at)
        found = jnp.where(hit, 1, found)
        return (cum2 + ct, found, jch, cum2_at)

    _, _, jch, cum2_at = lax.fori_loop(0, blksz // L, ch_body,
                                       (cum0, z, z, cum0))

    sl = pl.ds(jch * L, L)
    a = ba0[sl] + ba1[sl]
    rev = lax.rev(a, (0,))
    cs = plsc.cumsum(rev)
    maskv = (_splat(cum2_at) + cs) >= _splat(rank_s)
    jv = plsc.all_reduce_ffs(maskv)
    jv = _splat(jv) if jv.ndim == 0 else jv
    iota = lax.iota(jnp.int32, L)
    above = jnp.where(iota < jv, rev, 0)
    c_above = _splat(cum2_at) + _splat(jnp.sum(above))
    binv = _splat(base_bin + jch * L + (L - 1)) - jv
    return binv, c_above


def _scan_par(hist_hbm, nbins, rank_s, s, ba0, ba1, stage, sums2d, shared_s,
              semA, semB):
    blksz = nbins // NS
    d0 = ba0.at[pl.ds(0, blksz)]
    d1 = ba1.at[pl.ds(0, blksz)]
    off = pl.multiple_of(s * blksz, blksz)
    pltpu.async_copy(hist_hbm.at[pl.ds(off, blksz)], d0, semA)
    pltpu.async_copy(hist_hbm.at[pl.ds(nbins + off, blksz)], d1, semB)
    pltpu.make_async_copy(hist_hbm.at[pl.ds(off, blksz)], d0, semA).wait()
    pltpu.make_async_copy(hist_hbm.at[pl.ds(nbins + off, blksz)], d1,
                          semB).wait()

    @plsc.parallel_loop(0, blksz // L, unroll=8,
                        carry=jnp.zeros((L,), jnp.int32))
    def av(t, acc):
        sl = pl.ds(t * L, L)
        return acc + ba0[sl] + ba1[sl]

    stage[...] = _splat(jnp.sum(av))
    pltpu.sync_copy(stage, shared_s.at[s])
    plsc.subcore_barrier()
    pltpu.sync_copy(shared_s, sums2d)
    plsc.subcore_barrier()
    iota = lax.iota(jnp.int32, L)
    diag = plsc.load_gather(sums2d, [iota, iota])
    revd = lax.rev(diag, (0,))
    csd = plsc.cumsum(revd)
    maskb = csd >= _splat(rank_s)
    jbv = plsc.all_reduce_ffs(maskb)
    jbv = _splat(jbv) if jbv.ndim == 0 else jbv
    cumb = _splat(jnp.sum(jnp.where(iota < jbv, revd, 0)))
    jblk = jnp.max(_splat(NS - 1) - jbv)
    cum_at = jnp.max(cumb)

    offh = pl.multiple_of(jblk * blksz, blksz)
    pltpu.async_copy(hist_hbm.at[pl.ds(offh, blksz)], d0, semA)
    pltpu.async_copy(hist_hbm.at[pl.ds(nbins + offh, blksz)], d1, semB)
    pltpu.make_async_copy(hist_hbm.at[pl.ds(offh, blksz)], d0, semA).wait()
    pltpu.make_async_copy(hist_hbm.at[pl.ds(nbins + offh, blksz)], d1,
                          semB).wait()
    return _scan_block(ba0, ba1, blksz, jblk * blksz, cum_at, rank_s)


def _merge(hist2d, shared2d, idx, tmp, out_hbm, s, c, nbins):
    nr = nbins // 128

    @pl.when(s == 0)
    def _():
        pltpu.sync_copy(hist2d, shared2d)

    plsc.subcore_barrier()

    @pl.when(s != 0)
    def _():
        for r in range(nr // 128):
            pltpu.sync_copy(hist2d.at[pl.ds(r * 128, 128)],
                            shared2d.at[idx.at[r]], add=True)

    plsc.subcore_barrier()

    rpt = nr // NS

    def rb(r, _):
        pltpu.sync_copy(shared2d.at[s * rpt + r], tmp.at[pl.ds(r * 128, 128)])
        return 0

    lax.fori_loop(0, rpt, rb, 0)
    slc = nbins // NS
    pltpu.sync_copy(tmp, out_hbm.at[pl.ds(c * nbins + s * slc, slc)])


@functools.partial(
    pl.kernel,
    out_type=jax.ShapeDtypeStruct((NC * BA,), jnp.int32),
    mesh=_mesh,
    compiler_params=_params,
    scratch_types=[
        pltpu.VMEM((8, DK), jnp.float32),
        pltpu.VMEM((8, DK), jnp.float32),
        pltpu.VMEM((2 * BA // 128, 128), jnp.int32),
        pltpu.VMEM((2 * BA // 128 // 128, 128), jnp.int32),
        pltpu.VMEM((BA // NS,), jnp.int32),
        pltpu.VMEM((BA // NS,), jnp.int32),
        pltpu.VMEM((BA // NS,), jnp.int32),
        pltpu.VMEM_SHARED((2 * BA // 128, 128), jnp.int32),
        pltpu.SemaphoreType.DMA,
        pltpu.SemaphoreType.DMA,
    ],
)
def _k1(x_hbm, out_hbm, buf0, buf1, hist, idx, wa, wb, tmp, shared,
        sem0, sem1):
    c = lax.axis_index("c")
    s = lax.axis_index("s")
    _zero2d(hist, 2 * BA)
    _fill_idx(idx, 2 * BA // 128 // 128)
    ones = jnp.ones((L,), jnp.int32)

    def process(buf):
        for r in range(8):
            @plsc.parallel_loop(0, DK // L, unroll=8)
            def _(k):
                v = buf[r, pl.ds(k * L, L)]
                u = lax.bitcast_convert_type(v, jnp.int32)
                row = lax.shift_right_logical(u, 23)
                col = lax.shift_right_logical(u, 16) & 0x7F
                plsc.addupdate_scatter(hist, [row, col], ones)

    _db_pass(x_hbm, c, s, 8, buf0, buf1, sem0, sem1, process)

    @pl.when(s == 0)
    def _():
        pltpu.sync_copy(hist, shared)

    plsc.subcore_barrier()

    @pl.when(s != 0)
    def _():
        for r in range(2 * BA // 128 // 128):
            pltpu.sync_copy(hist.at[pl.ds(r * 128, 128)],
                            shared.at[idx.at[r]], add=True)

    plsc.subcore_barrier()

    def rb(r, _):
        pltpu.sync_copy(shared.at[s * 16 + r], wa.at[pl.ds(r * 128, 128)])
        pltpu.sync_copy(shared.at[s * 16 + r + 256],
                        wb.at[pl.ds(r * 128, 128)])
        return 0

    lax.fori_loop(0, 16, rb, 0)

    @plsc.parallel_loop(0, (BA // NS) // L, unroll=8)
    def _(t):
        sl = pl.ds(t * L, L)
        tmp[sl] = wa[sl] + wb[sl]

    slc = BA // NS
    pltpu.sync_copy(tmp, out_hbm.at[pl.ds(c * BA + s * slc, slc)])


@functools.partial(
    pl.kernel,
    out_type=jax.ShapeDtypeStruct((NC * BB,), jnp.int32),
    mesh=_mesh,
    compiler_params=_params,
    scratch_types=[
        pltpu.VMEM((8, DK), jnp.float32),
        pltpu.VMEM((8, DK), jnp.float32),
        pltpu.VMEM((BB // 128, 128), jnp.int32),
        pltpu.VMEM((BB // 128 // 128, 128), jnp.int32),
        pltpu.VMEM((BA // NS,), jnp.int32),
        pltpu.VMEM((BA // NS,), jnp.int32),
        pltpu.VMEM((L,), jnp.int32),
        pltpu.VMEM((NS, L), jnp.int32),
        pltpu.VMEM((BB // NS,), jnp.int32),
        pltpu.VMEM_SHARED((BB // 128, 128), jnp.int32),
        pltpu.VMEM_SHARED((NS, L), jnp.int32),
        pltpu.SemaphoreType.DMA,
        pltpu.SemaphoreType.DMA,
    ],
)
def _k2(x_hbm, ha_hbm, out_hbm, buf0, buf1, hist, idx, ba0, ba1, stage,
        sums2d, tmp, shared, shared_s, sem0, sem1):
    c = lax.axis_index("c")
    s = lax.axis_index("s")
    binv, _ = _scan_par(ha_hbm, BA, jnp.int32(KP), s, ba0, ba1, stage,
                        sums2d, shared_s, sem0, sem1)
    bhi = lax.shift_left(binv, 16)
    _zero2d(hist, BB)
    _fill_idx(idx, BB // 128 // 128)
    ones = jnp.ones((L,), jnp.int32)

    def process(buf):
        for r in range(8):
            @plsc.parallel_loop(0, DK // L, unroll=8)
            def _(k):
                v = buf[r, pl.ds(k * L, L)]
                u = lax.bitcast_convert_type(v, jnp.int32)
                row = lax.shift_right_logical(u, 7) & 0x1FF
                col = u & 0x7F
                plsc.addupdate_scatter(hist, [row, col], ones,
                                       mask=(u & 0x7FFF0000) == bhi)

    _db_pass(x_hbm, c, s, 8, buf0, buf1, sem0, sem1, process)
    _merge(hist, shared, idx, tmp, out_hbm, s, c, BB)


@functools.partial(
    pl.kernel,
    out_type=jax.ShapeDtypeStruct((L,), jnp.float32),
    mesh=_mesh,
    compiler_params=_params,
    scratch_types=[
        pltpu.VMEM((BB // NS,), jnp.int32),
        pltpu.VMEM((BB // NS,), jnp.int32),
        pltpu.VMEM((L,), jnp.int32),
        pltpu.VMEM((NS, L), jnp.int32),
        pltpu.VMEM((L,), jnp.float32),
        pltpu.VMEM_SHARED((NS, L), jnp.int32),
        pltpu.SemaphoreType.DMA,
        pltpu.SemaphoreType.DMA,
    ],
)
def _k3(ha_hbm, hb_hbm, out_hbm, ba0, ba1, stage, sums2d, ovec, shared_s,
        semA, semB):
    c = lax.axis_index("c")
    s = lax.axis_index("s")
    binv, c_above = _scan_par(ha_hbm, BA, jnp.int32(KP), s, ba0, ba1, stage,
                              sums2d, shared_s, semA, semB)
    r = KP - jnp.max(c_above)
    lov, _ = _scan_par(hb_hbm, BB, r, s, ba0, ba1, stage, sums2d, shared_s,
                       semA, semB)

    @pl.when(jnp.logical_and(c == 0, s == 0))
    def _():
        bits = lax.shift_left(binv, 16) | lov
        ovec[...] = lax.bitcast_convert_type(bits, jnp.float32)
        pltpu.sync_copy(ovec, out_hbm)


def kernel(input):
    ha = _k1(input)
    hb = _k2(input, ha)
    out = _k3(ha, hb)
    return out[:1]


# speedup vs baseline: 1.0195x; 1.0195x over previous
import functools

import jax
import jax.numpy as jnp
from jax import lax
from jax.experimental import pallas as pl
from jax.experimental.pallas import tpu as pltpu
from jax.experimental.pallas import tpu_sc as plsc

B0 = 2
NR = 4096
DK = 2048
N = B0 * NR * DK
KTH = int(0.9999 * N)
KP = N - KTH + 1
NC = 2
NS = 16
L = 16
RPT = NR // NS
BA = 1 << 15
BB = 1 << 16

_mesh = plsc.VectorSubcoreMesh(core_axis_name="c", subcore_axis_name="s")
_params = pltpu.CompilerParams(needs_layout_passes=False)


def _splat(x):
    return jnp.broadcast_to(x, (L,))


def _zero2d(ref, nbins):
    z = jnp.zeros((L,), jnp.int32)

    @plsc.parallel_loop(0, nbins // L, unroll=8)
    def _(t):
        ref[t // 8, pl.ds((t % 8) * L, L)] = z


def _fill_idx(idx, ngrp):
    iota = lax.iota(jnp.int32, L)

    def b(t, _):
        r = t // 8
        k = t % 8
        idx[r, pl.ds(k * L, L)] = r * 128 + k * L + iota
        return 0

    lax.fori_loop(0, ngrp * 8, b, 0)


def _db_pass(x_hbm, c, s, rch, buf0, buf1, sem0, sem1, process):
    r0 = s * RPT
    nch = RPT // rch

    def start(ci, buf, sem):
        pltpu.async_copy(x_hbm.at[c, pl.ds(r0 + ci * rch, rch), :], buf, sem)

    def wait(ci, buf, sem):
        pltpu.make_async_copy(
            x_hbm.at[c, pl.ds(r0 + ci * rch, rch), :], buf, sem).wait()

    start(0, buf0, sem0)

    def body(i, _):
        ci0 = 2 * i
        start(ci0 + 1, buf1, sem1)
        wait(ci0, buf0, sem0)
        process(buf0)

        @pl.when(ci0 + 2 < nch)
        def _():
            start(ci0 + 2, buf0, sem0)

        wait(ci0 + 1, buf1, sem1)
        process(buf1)
        return 0

    lax.fori_loop(0, nch // 2, body, 0)


def _scan_block(ba0, ba1, blksz, base_bin, cum0, rank_s):
    z = jnp.int32(0)

    def ch_body(i, carry):
        cum2, found, jch, cum2_at = carry
        t = blksz // L - 1 - i
        sl = pl.ds(t * L, L)
        ct = jnp.sum(ba0[sl] + ba1[sl])
        hit = jnp.logical_and(found == 0, cum2 + ct >= rank_s)
        jch = jnp.where(hit, t, jch)
        cum2_at = jnp.where(hit, cum2, cum2_at)
        found = jnp.where(hit, 1, found)
        return (cum2 + ct, found, jch, cum2_at)

    _, _, jch, cum2_at = lax.fori_loop(0, blksz // L, ch_body,
                                       (cum0, z, z, cum0))

    sl = pl.ds(jch * L, L)
    a = ba0[sl] + ba1[sl]
    rev = lax.rev(a, (0,))
    cs = plsc.cumsum(rev)
    maskv = (_splat(cum2_at) + cs) >= _splat(rank_s)
    jv = plsc.all_reduce_ffs(maskv)
    jv = _splat(jv) if jv.ndim == 0 else jv
    iota = lax.iota(jnp.int32, L)
    above = jnp.where(iota < jv, rev, 0)
    c_above = _splat(cum2_at) + _splat(jnp.sum(above))
    binv = _splat(base_bin + jch * L + (L - 1)) - jv
    return binv, c_above


def _scan_par(hist_hbm, nbins, rank_s, s, ba0, ba1, stage, sums2d, shared_s,
              semA, semB):
    blksz = nbins // NS
    d0 = ba0.at[pl.ds(0, blksz)]
    d1 = ba1.at[pl.ds(0, blksz)]
    off = pl.multiple_of(s * blksz, blksz)
    pltpu.async_copy(hist_hbm.at[pl.ds(off, blksz)], d0, semA)
    pltpu.async_copy(hist_hbm.at[pl.ds(nbins + off, blksz)], d1, semB)
    pltpu.make_async_copy(hist_hbm.at[pl.ds(off, blksz)], d0, semA).wait()
    pltpu.make_async_copy(hist_hbm.at[pl.ds(nbins + off, blksz)], d1,
                          semB).wait()

    @plsc.parallel_loop(0, blksz // L, unroll=8,
                        carry=jnp.zeros((L,), jnp.int32))
    def av(t, acc):
        sl = pl.ds(t * L, L)
        return acc + ba0[sl] + ba1[sl]

    stage[...] = _splat(jnp.sum(av))
    pltpu.sync_copy(stage, shared_s.at[s])
    plsc.subcore_barrier()
    pltpu.sync_copy(shared_s, sums2d)
    plsc.subcore_barrier()
    iota = lax.iota(jnp.int32, L)
    diag = plsc.load_gather(sums2d, [iota, iota])
    revd = lax.rev(diag, (0,))
    csd = plsc.cumsum(revd)
    maskb = csd >= _splat(rank_s)
    jbv = plsc.all_reduce_ffs(maskb)
    jbv = _splat(jbv) if jbv.ndim == 0 else jbv
    cumb = _splat(jnp.sum(jnp.where(iota < jbv, revd, 0)))
    jblk = jnp.max(_splat(NS - 1) - jbv)
    cum_at = jnp.max(cumb)

    offh = pl.multiple_of(jblk * blksz, blksz)
    pltpu.async_copy(hist_hbm.at[pl.ds(offh, blksz)], d0, semA)
    pltpu.async_copy(hist_hbm.at[pl.ds(nbins + offh, blksz)], d1, semB)
    pltpu.make_async_copy(hist_hbm.at[pl.ds(offh, blksz)], d0, semA).wait()
    pltpu.make_async_copy(hist_hbm.at[pl.ds(nbins + offh, blksz)], d1,
                          semB).wait()
    return _scan_block(ba0, ba1, blksz, jblk * blksz, cum_at, rank_s)


def _merge(hist2d, shared2d, idx, tmp, out_hbm, s, c, nbins):
    nr = nbins // 128

    @pl.when(s == 0)
    def _():
        pltpu.sync_copy(hist2d, shared2d)

    plsc.subcore_barrier()

    @pl.when(s != 0)
    def _():
        for r in range(nr // 128):
            pltpu.sync_copy(hist2d.at[pl.ds(r * 128, 128)],
                            shared2d.at[idx.at[r]], add=True)

    plsc.subcore_barrier()

    rpt = nr // NS

    def rb(r, _):
        pltpu.sync_copy(shared2d.at[s * rpt + r], tmp.at[pl.ds(r * 128, 128)])
        return 0

    lax.fori_loop(0, rpt, rb, 0)
    slc = nbins // NS
    pltpu.sync_copy(tmp, out_hbm.at[pl.ds(c * nbins + s * slc, slc)])


@functools.partial(
    pl.kernel,
    out_type=jax.ShapeDtypeStruct((NC * BA,), jnp.int32),
    mesh=_mesh,
    compiler_params=_params,
    scratch_types=[
        pltpu.VMEM((16, DK), jnp.float32),
        pltpu.VMEM((16, DK), jnp.float32),
        pltpu.VMEM((BA // 128, 128), jnp.int32),
        pltpu.VMEM((BA // 128 // 128, 128), jnp.int32),
        pltpu.VMEM((BA // NS,), jnp.int32),
        pltpu.VMEM_SHARED((BA // 128, 128), jnp.int32),
        pltpu.SemaphoreType.DMA,
        pltpu.SemaphoreType.DMA,
    ],
)
def _k1(x_hbm, out_hbm, buf0, buf1, hist, idx, tmp, shared, sem0, sem1):
    c = lax.axis_index("c")
    s = lax.axis_index("s")
    _zero2d(hist, BA)
    _fill_idx(idx, BA // 128 // 128)
    ones = jnp.ones((L,), jnp.int32)

    def process(buf):
        for r in range(16):
            @plsc.parallel_loop(0, DK // L, unroll=8)
            def _(k):
                v = buf[r, pl.ds(k * L, L)]
                u = lax.bitcast_convert_type(v, jnp.int32)
                row = lax.shift_right_logical(u, 23) & 0xFF
                col = lax.shift_right_logical(u, 16) & 0x7F
                plsc.addupdate_scatter(hist, [row, col], ones)

    _db_pass(x_hbm, c, s, 16, buf0, buf1, sem0, sem1, process)
    _merge(hist, shared, idx, tmp, out_hbm, s, c, BA)


@functools.partial(
    pl.kernel,
    out_type=jax.ShapeDtypeStruct((NC * BB,), jnp.int32),
    mesh=_mesh,
    compiler_params=_params,
    scratch_types=[
        pltpu.VMEM((8, DK), jnp.float32),
        pltpu.VMEM((8, DK), jnp.float32),
        pltpu.VMEM((BB // 128, 128), jnp.int32),
        pltpu.VMEM((BB // 128 // 128, 128), jnp.int32),
        pltpu.VMEM((BA // NS,), jnp.int32),
        pltpu.VMEM((BA // NS,), jnp.int32),
        pltpu.VMEM((L,), jnp.int32),
        pltpu.VMEM((NS, L), jnp.int32),
        pltpu.VMEM((BB // NS,), jnp.int32),
        pltpu.VMEM_SHARED((BB // 128, 128), jnp.int32),
        pltpu.VMEM_SHARED((NS, L), jnp.int32),
        pltpu.SemaphoreType.DMA,
        pltpu.SemaphoreType.DMA,
    ],
)
def _k2(x_hbm, ha_hbm, out_hbm, buf0, buf1, hist, idx, ba0, ba1, stage,
        sums2d, tmp, shared, shared_s, sem0, sem1):
    c = lax.axis_index("c")
    s = lax.axis_index("s")
    binv, _ = _scan_par(ha_hbm, BA, jnp.int32(KP), s, ba0, ba1, stage,
                        sums2d, shared_s, sem0, sem1)
    bhi = lax.shift_left(binv, 16)
    _zero2d(hist, BB)
    _fill_idx(idx, BB // 128 // 128)
    ones = jnp.ones((L,), jnp.int32)

    def process(buf):
        for r in range(8):
            @plsc.parallel_loop(0, DK // L, unroll=8)
            def _(k):
                v = buf[r, pl.ds(k * L, L)]
                u = lax.bitcast_convert_type(v, jnp.int32)
                row = lax.shift_right_logical(u, 7) & 0x1FF
                col = u & 0x7F
                plsc.addupdate_scatter(hist, [row, col], ones,
                                       mask=(u & 0x7FFF0000) == bhi)

    _db_pass(x_hbm, c, s, 8, buf0, buf1, sem0, sem1, process)
    _merge(hist, shared, idx, tmp, out_hbm, s, c, BB)


@functools.partial(
    pl.kernel,
    out_type=jax.ShapeDtypeStruct((L,), jnp.float32),
    mesh=_mesh,
    compiler_params=_params,
    scratch_types=[
        pltpu.VMEM((BB // NS,), jnp.int32),
        pltpu.VMEM((BB // NS,), jnp.int32),
        pltpu.VMEM((L,), jnp.int32),
        pltpu.VMEM((NS, L), jnp.int32),
        pltpu.VMEM((L,), jnp.float32),
        pltpu.VMEM_SHARED((NS, L), jnp.int32),
        pltpu.SemaphoreType.DMA,
        pltpu.SemaphoreType.DMA,
    ],
)
def _k3(ha_hbm, hb_hbm, out_hbm, ba0, ba1, stage, sums2d, ovec, shared_s,
        semA, semB):
    c = lax.axis_index("c")
    s = lax.axis_index("s")
    binv, c_above = _scan_par(ha_hbm, BA, jnp.int32(KP), s, ba0, ba1, stage,
                              sums2d, shared_s, semA, semB)
    r = KP - jnp.max(c_above)
    lov, _ = _scan_par(hb_hbm, BB, r, s, ba0, ba1, stage, sums2d, shared_s,
                       semA, semB)

    @pl.when(jnp.logical_and(c == 0, s == 0))
    def _():
        bits = lax.shift_left(binv, 16) | lov
        ovec[...] = lax.bitcast_convert_type(bits, jnp.float32)
        pltpu.sync_copy(ovec, out_hbm)


def kernel(input):
    ha = _k1(input)
    hb = _k2(input, ha)
    out = _k3(ha, hb)
    return out[:1]
